# Initial kernel scaffold; baseline (speedup 1.0000x reference)
import jax
import jax.numpy as jnp
from jax.experimental import pallas as pl


def kernel(feat, coords, knn_idx, W, b):
    raise NotImplementedError("write your pallas kernel here")


# speedup vs baseline: 5.7032x; 5.7032x over previous
import functools

import jax
import jax.numpy as jnp
from jax import lax
from jax.experimental import pallas as pl
from jax.experimental.pallas import tpu as pltpu
from jax.experimental.pallas import tpu_sc as plsc

N = 10000
C = 128
K = 32
HIDDEN = 128
CP = 16
BN = 4
NBLK = N // BN
NW = 32
TMAX = (NBLK + NW - 1) // NW
LANES = 16
CCHUNK = C // LANES


def _sc_gather_body(knn_hbm, feat_hbm, cpad_hbm, aggsum_hbm, stats_hbm,
                    idx_v, rows_v, crows_v, agg_v, stats_v, sem, sem2):
  cid = lax.axis_index("c")
  sid = lax.axis_index("s")
  wid = sid * 2 + cid

  def step(t, _):
    b = t * NW + wid

    @pl.when(b < NBLK)
    def _():
      pltpu.sync_copy(knn_hbm.at[pl.ds(b, 1)], idx_v)
      cp1 = pltpu.make_async_copy(feat_hbm.at[idx_v.at[0]], rows_v, sem)
      cp2 = pltpu.make_async_copy(cpad_hbm.at[idx_v.at[0]], crows_v, sem2)
      cp1.start()
      cp2.start()
      cp1.wait()
      cp2.wait()

      for n in range(BN):
        def red(j, carry):
          accs = carry[:CCHUNK]
          cs, cq = carry[CCHUNK], carry[CCHUNK + 1]
          r = n * K + j
          new_accs = tuple(
              accs[c] + rows_v[r, pl.ds(c * LANES, LANES)]
              for c in range(CCHUNK))
          crow = crows_v[r, :]
          return new_accs + (cs + crow, cq + crow * crow)

        zero = jnp.zeros((LANES,), jnp.float32)
        init = tuple(zero for _ in range(CCHUNK + 2))
        out = lax.fori_loop(0, K, red, init)
        for c in range(CCHUNK):
          agg_v[n, pl.ds(c * LANES, LANES)] = out[c]
        stats_v[n, pl.ds(0, LANES)] = out[CCHUNK]
        stats_v[n, pl.ds(LANES, LANES)] = out[CCHUNK + 1]

      pltpu.sync_copy(agg_v, aggsum_hbm.at[pl.ds(b * BN, BN)])
      pltpu.sync_copy(stats_v, stats_hbm.at[pl.ds(b * BN, BN)])

    return _

  lax.fori_loop(0, TMAX, step, None)


def _sc_gather(knn2d, feat, cpad):
  mesh = plsc.VectorSubcoreMesh(core_axis_name="c", subcore_axis_name="s")
  return pl.kernel(
      _sc_gather_body,
      out_type=(
          jax.ShapeDtypeStruct((N, C), jnp.float32),
          jax.ShapeDtypeStruct((N, 2 * LANES), jnp.float32),
      ),
      mesh=mesh,
      compiler_params=pltpu.CompilerParams(use_tc_tiling_on_sc=False),
      scratch_types=[
          pltpu.VMEM((1, BN * K), jnp.int32),
          pltpu.VMEM((BN * K, C), jnp.float32),
          pltpu.VMEM((BN * K, CP), jnp.float32),
          pltpu.VMEM((BN, C), jnp.float32),
          pltpu.VMEM((BN, 2 * LANES), jnp.float32),
          pltpu.SemaphoreType.DMA,
          pltpu.SemaphoreType.DMA,
      ],
  )(knn2d, feat, cpad)


def _tc_dense_body(feat_ref, agg_ref, stats_ref, cp_ref,
                   w1_ref, w2_ref, wm_ref, ws_ref, b_ref, out_ref):
  inv_k = 1.0 / K
  st = stats_ref[...]
  m1 = st[:, 0:LANES] * inv_k
  m2 = st[:, LANES:2 * LANES] * inv_k
  rm = m1 - cp_ref[...]
  rs = jnp.sqrt(jnp.maximum(m2 - m1 * m1, 0.0))
  acc = jnp.dot(feat_ref[...], w1_ref[...], preferred_element_type=jnp.float32)
  acc += jnp.dot(agg_ref[...], w2_ref[...], preferred_element_type=jnp.float32)
  acc += jnp.dot(rm, wm_ref[...], preferred_element_type=jnp.float32)
  acc += jnp.dot(rs, ws_ref[...], preferred_element_type=jnp.float32)
  acc += b_ref[...]
  out_ref[...] = acc * jax.nn.sigmoid(acc)


def _tc_dense(feat, aggsum, stats, cpad, w1, w2s, wm, ws, b2):
  R = 1000
  grid = (N // R,)
  return pl.pallas_call(
      _tc_dense_body,
      grid=grid,
      in_specs=[
          pl.BlockSpec((R, C), lambda i: (i, 0)),
          pl.BlockSpec((R, C), lambda i: (i, 0)),
          pl.BlockSpec((R, 2 * LANES), lambda i: (i, 0)),
          pl.BlockSpec((R, CP), lambda i: (i, 0)),
          pl.BlockSpec((C, HIDDEN), lambda i: (0, 0)),
          pl.BlockSpec((C, HIDDEN), lambda i: (0, 0)),
          pl.BlockSpec((CP, HIDDEN), lambda i: (0, 0)),
          pl.BlockSpec((CP, HIDDEN), lambda i: (0, 0)),
          pl.BlockSpec((1, HIDDEN), lambda i: (0, 0)),
      ],
      out_specs=pl.BlockSpec((R, HIDDEN), lambda i: (i, 0)),
      out_shape=jax.ShapeDtypeStruct((N, HIDDEN), jnp.float32),
  )(feat, aggsum, stats, cpad, w1, w2s, wm, ws, b2)


def kernel(feat, coords, knn_idx, W, b):
  knn2d = knn_idx.astype(jnp.int32).reshape(NBLK, BN * K)
  cpad = jnp.zeros((N, CP), jnp.float32).at[:, :3].set(coords)
  aggsum, stats = _sc_gather(knn2d, feat, cpad)

  w1 = W[0:C]
  w2s = W[C:2 * C] * (1.0 / K)
  wm = jnp.zeros((CP, HIDDEN), jnp.float32).at[0:3].set(W[2 * C:2 * C + 3])
  ws = jnp.zeros((CP, HIDDEN), jnp.float32).at[0:3].set(W[2 * C + 3:2 * C + 6])
  b2 = b.reshape(1, HIDDEN)
  return _tc_dense(feat, aggsum, stats, cpad, w1, w2s, wm, ws, b2)


# speedup vs baseline: 9.9456x; 1.7439x over previous
import jax
import jax.numpy as jnp
from jax import lax
from jax.experimental import pallas as pl
from jax.experimental.pallas import tpu as pltpu
from jax.experimental.pallas import tpu_sc as plsc

N = 10000
C = 128
K = 32
HIDDEN = 128
CP = 16
BN = 4
NBLK = N // BN
NW = 32
TL = 80
NBLK_PAD = NW * TL
LANES = 16
CCHUNK = C // LANES


def _sc_gather_body(knn_hbm, feat_hbm, cpad_hbm, aggsum_hbm, stats_hbm,
                    idx_all, rows0, rows1, crows0, crows1,
                    agg0, agg1, st0, st1, semg0, semg1, semo0, semo1):
  cid = lax.axis_index("c")
  sid = lax.axis_index("s")
  wid = sid * 2 + cid
  base = wid * TL

  rows_b = (rows0, rows1)
  crows_b = (crows0, crows1)
  agg_b = (agg0, agg1)
  st_b = (st0, st1)
  semg = (semg0, semg1)
  semo = (semo0, semo1)

  pltpu.sync_copy(knn_hbm.at[pl.ds(base, TL)], idx_all)

  def gathers(t, p):
    return (
        pltpu.make_async_copy(feat_hbm.at[idx_all.at[t]], rows_b[p], semg[p]),
        pltpu.make_async_copy(cpad_hbm.at[idx_all.at[t]], crows_b[p], semg[p]),
    )

  def out_copies(t, p):
    return (
        pltpu.make_async_copy(agg_b[p], aggsum_hbm.at[pl.ds((base + t) * BN, BN)],
                              semo[p]),
        pltpu.make_async_copy(st_b[p], stats_hbm.at[pl.ds((base + t) * BN, BN)],
                              semo[p]),
    )

  for cp in gathers(0, 0):
    cp.start()

  def outer(g, _):
    for phase in range(2):
      t = g * 2 + phase
      tn = t + 1

      @pl.when((tn < TL) & (base + tn < NBLK))
      def _():
        for cp in gathers(tn, 1 - phase):
          cp.start()

      @pl.when(base + t < NBLK)
      def _():
        @pl.when(t >= 2)
        def _():
          for cp in out_copies(t - 2, phase):
            cp.wait()

        for cp in gathers(t, phase):
          cp.wait()

        rows_v = rows_b[phase]
        crows_v = crows_b[phase]
        for n in range(BN):
          def red(j, carry):
            accs = carry[:CCHUNK]
            cs, cq = carry[CCHUNK], carry[CCHUNK + 1]
            r = n * K + j
            new_accs = tuple(
                accs[c] + rows_v[r, pl.ds(c * LANES, LANES)]
                for c in range(CCHUNK))
            crow = crows_v[r, :]
            return new_accs + (cs + crow, cq + crow * crow)

          zero = jnp.zeros((LANES,), jnp.float32)
          init = tuple(zero for _ in range(CCHUNK + 2))
          out = lax.fori_loop(0, K, red, init)
          for c in range(CCHUNK):
            agg_b[phase][n, pl.ds(c * LANES, LANES)] = out[c]
          st_b[phase][n, pl.ds(0, LANES)] = out[CCHUNK]
          st_b[phase][n, pl.ds(LANES, LANES)] = out[CCHUNK + 1]

        for cp in out_copies(t, phase):
          cp.start()

    return _

  lax.fori_loop(0, TL // 2, outer, None)

  for p in range(2):
    for cp in out_copies(0, p):
      cp.wait()


def _sc_gather(knn2d, feat, cpad):
  mesh = plsc.VectorSubcoreMesh(core_axis_name="c", subcore_axis_name="s")
  return pl.kernel(
      _sc_gather_body,
      out_type=(
          jax.ShapeDtypeStruct((N, C), jnp.float32),
          jax.ShapeDtypeStruct((N, 2 * LANES), jnp.float32),
      ),
      mesh=mesh,
      compiler_params=pltpu.CompilerParams(use_tc_tiling_on_sc=False),
      scratch_types=[
          pltpu.VMEM((TL, BN * K), jnp.int32),
          pltpu.VMEM((BN * K, C), jnp.float32),
          pltpu.VMEM((BN * K, C), jnp.float32),
          pltpu.VMEM((BN * K, CP), jnp.float32),
          pltpu.VMEM((BN * K, CP), jnp.float32),
          pltpu.VMEM((BN, C), jnp.float32),
          pltpu.VMEM((BN, C), jnp.float32),
          pltpu.VMEM((BN, 2 * LANES), jnp.float32),
          pltpu.VMEM((BN, 2 * LANES), jnp.float32),
          pltpu.SemaphoreType.DMA,
          pltpu.SemaphoreType.DMA,
          pltpu.SemaphoreType.DMA,
          pltpu.SemaphoreType.DMA,
      ],
  )(knn2d, feat, cpad)


def _tc_dense_body(feat_ref, agg_ref, stats_ref, cp_ref,
                   w1_ref, w2_ref, wm_ref, ws_ref, b_ref, out_ref):
  inv_k = 1.0 / K
  st = stats_ref[...]
  m1 = st[:, 0:LANES] * inv_k
  m2 = st[:, LANES:2 * LANES] * inv_k
  rm = m1 - cp_ref[...]
  rs = jnp.sqrt(jnp.maximum(m2 - m1 * m1, 0.0))
  acc = jnp.dot(feat_ref[...], w1_ref[...], preferred_element_type=jnp.float32)
  acc += jnp.dot(agg_ref[...], w2_ref[...], preferred_element_type=jnp.float32)
  acc += jnp.dot(rm, wm_ref[...], preferred_element_type=jnp.float32)
  acc += jnp.dot(rs, ws_ref[...], preferred_element_type=jnp.float32)
  acc += b_ref[...]
  out_ref[...] = acc * jax.nn.sigmoid(acc)


def _tc_dense(feat, aggsum, stats, cpad, w1, w2s, wm, ws, b2):
  R = 1000
  grid = (N // R,)
  return pl.pallas_call(
      _tc_dense_body,
      grid=grid,
      in_specs=[
          pl.BlockSpec((R, C), lambda i: (i, 0)),
          pl.BlockSpec((R, C), lambda i: (i, 0)),
          pl.BlockSpec((R, 2 * LANES), lambda i: (i, 0)),
          pl.BlockSpec((R, CP), lambda i: (i, 0)),
          pl.BlockSpec((C, HIDDEN), lambda i: (0, 0)),
          pl.BlockSpec((C, HIDDEN), lambda i: (0, 0)),
          pl.BlockSpec((CP, HIDDEN), lambda i: (0, 0)),
          pl.BlockSpec((CP, HIDDEN), lambda i: (0, 0)),
          pl.BlockSpec((1, HIDDEN), lambda i: (0, 0)),
      ],
      out_specs=pl.BlockSpec((R, HIDDEN), lambda i: (i, 0)),
      out_shape=jax.ShapeDtypeStruct((N, HIDDEN), jnp.float32),
  )(feat, aggsum, stats, cpad, w1, w2s, wm, ws, b2)


def kernel(feat, coords, knn_idx, W, b):
  knn2d = jnp.zeros((NBLK_PAD, BN * K), jnp.int32).at[:NBLK].set(
      knn_idx.astype(jnp.int32).reshape(NBLK, BN * K))
  cpad = jnp.zeros((N, CP), jnp.float32).at[:, :3].set(coords)
  aggsum, stats = _sc_gather(knn2d, feat, cpad)

  w1 = W[0:C]
  w2s = W[C:2 * C] * (1.0 / K)
  wm = jnp.zeros((CP, HIDDEN), jnp.float32).at[0:3].set(W[2 * C:2 * C + 3])
  ws = jnp.zeros((CP, HIDDEN), jnp.float32).at[0:3].set(W[2 * C + 3:2 * C + 6])
  b2 = b.reshape(1, HIDDEN)
  return _tc_dense(feat, aggsum, stats, cpad, w1, w2s, wm, ws, b2)
